# baseline (speedup 1.0000x reference)
import functools

import jax
import jax.numpy as jnp
from jax import lax
from jax.experimental import pallas as pl
from jax.experimental.pallas import tpu as pltpu
from jax.experimental.pallas import tpu_sc as plsc

_INPUT_DIM = 30
_N_CAT = 26
_VOCAB = 100000
_EMB = 16
_BATCH = 4096
_N_CONT = _INPUT_DIM - _N_CAT
_TAB_ROWS = _N_CAT * _EMB
_OUT_ROWS = _TAB_ROWS + _N_CONT
_NC = 2
_NS = 16
_NW = _NC * _NS
_RPW = _TAB_ROWS // _NW
_L = 16
_H = 49920
_TAIL = _VOCAB - 2 * _H


def _body(xt_hbm, tt_hbm, tl_hbm, out_hbm, rowa, rowb, tailb, colb, outb,
          sema, semb, semt):
    c = lax.axis_index("c")
    s = lax.axis_index("s")
    w = s * _NC + c
    r0 = w * _RPW

    def fe(r):
        return r // _EMB, lax.rem(r, _EMB)

    def fetch_lo(r):
        f, e = fe(r)
        pltpu.async_copy(tt_hbm.at[f].at[e].at[pl.ds(0, _H)], rowa, sema)

    def fetch_hi(r):
        f, e = fe(r)
        pltpu.async_copy(tt_hbm.at[f].at[e].at[pl.ds(_H, _H)], rowb, semb)

    def fetch_tail(r):
        f, e = fe(r)
        pltpu.async_copy(tl_hbm.at[f].at[e], tailb, semt)

    f_first, _ = fe(r0)
    pltpu.sync_copy(xt_hbm.at[f_first], colb)
    fetch_lo(r0)
    fetch_hi(r0)
    fetch_tail(r0)

    def do_r(j, prev_f):
        r = r0 + j
        f, _ = fe(r)

        @pl.when(f != prev_f)
        def _():
            pltpu.sync_copy(xt_hbm.at[f], colb)

        pltpu.make_async_copy(tt_hbm.at[0].at[0].at[pl.ds(0, _H)],
                              rowa, sema).wait()

        def pass_lo(i, cc):
            idx = colb[pl.ds(i * _L, _L)]
            m = idx < _H
            g = plsc.load_gather(rowa, [jnp.minimum(idx, _H - 1)], mask=m)
            outb[pl.ds(i * _L, _L)] = g
            return cc
        lax.fori_loop(0, _BATCH // _L, pass_lo, 0, unroll=8)

        @pl.when(j + 1 < _RPW)
        def _():
            fetch_lo(r + 1)

        pltpu.make_async_copy(tt_hbm.at[0].at[0].at[pl.ds(0, _H)],
                              rowb, semb).wait()

        def pass_hi(i, cc):
            idx = colb[pl.ds(i * _L, _L)]
            m = (idx >= _H) & (idx < 2 * _H)
            i2 = jnp.minimum(jnp.maximum(idx - _H, 0), _H - 1)
            g = plsc.load_gather(rowb, [i2], mask=m)
            prev = outb[pl.ds(i * _L, _L)]
            outb[pl.ds(i * _L, _L)] = jnp.where(m, g, prev)
            return cc
        lax.fori_loop(0, _BATCH // _L, pass_hi, 0, unroll=8)

        @pl.when(j + 1 < _RPW)
        def _():
            fetch_hi(r + 1)

        pltpu.make_async_copy(tl_hbm.at[0].at[0], tailb, semt).wait()

        def pass_tail(i, cc):
            idx = colb[pl.ds(i * _L, _L)]
            m = idx >= 2 * _H
            i2 = jnp.maximum(idx - 2 * _H, 0)
            g = plsc.load_gather(tailb, [i2], mask=m)
            prev = outb[pl.ds(i * _L, _L)]
            outb[pl.ds(i * _L, _L)] = jnp.where(m, g, prev)
            return cc
        lax.fori_loop(0, _BATCH // _L, pass_tail, 0, unroll=8)

        pltpu.sync_copy(outb, out_hbm.at[r])

        @pl.when(j + 1 < _RPW)
        def _():
            fetch_tail(r + 1)

        return f

    lax.fori_loop(0, _RPW, do_r, f_first)

    @pl.when(w < _N_CONT)
    def _():
        pltpu.sync_copy(xt_hbm.at[_N_CAT + w], colb)

        def chunk(i, cc):
            outb[pl.ds(i * _L, _L)] = colb[pl.ds(i * _L, _L)].astype(jnp.float32)
            return cc
        lax.fori_loop(0, _BATCH // _L, chunk, 0, unroll=8)
        pltpu.sync_copy(outb, out_hbm.at[_TAB_ROWS + w])


_emb_call = functools.partial(
    pl.kernel,
    mesh=plsc.VectorSubcoreMesh(core_axis_name="c", subcore_axis_name="s"),
    out_type=jax.ShapeDtypeStruct((_OUT_ROWS, _BATCH), jnp.float32),
    compiler_params=pltpu.CompilerParams(needs_layout_passes=False,
                                         use_tc_tiling_on_sc=True),
    scratch_types=[
        pltpu.VMEM((_H,), jnp.float32),
        pltpu.VMEM((_H,), jnp.float32),
        pltpu.VMEM((_TAIL,), jnp.float32),
        pltpu.VMEM((_BATCH,), jnp.int32),
        pltpu.VMEM((_BATCH,), jnp.float32),
        pltpu.SemaphoreType.DMA,
        pltpu.SemaphoreType.DMA,
        pltpu.SemaphoreType.DMA,
    ],
)(_body)


def kernel(x, tables):
    xt = x.T
    tt = tables.transpose(0, 2, 1)
    tl = tables[:, 2 * _H:, :].transpose(0, 2, 1)
    out_t = _emb_call(xt, tt, tl)
    return out_t.T


# speedup vs baseline: 1.2091x; 1.2091x over previous
import functools

import jax
import jax.numpy as jnp
from jax import lax
from jax.experimental import pallas as pl
from jax.experimental.pallas import tpu as pltpu
from jax.experimental.pallas import tpu_sc as plsc

_INPUT_DIM = 30
_N_CAT = 26
_VOCAB = 100000
_EMB = 16
_BATCH = 4096
_N_CONT = _INPUT_DIM - _N_CAT
_TAB_ROWS = _N_CAT * _EMB
_OUT_ROWS = _TAB_ROWS + _N_CONT
_NC = 2
_NS = 16
_NW = _NC * _NS
_RPW = _TAB_ROWS // _NW
_L = 16


def _body(xt_hbm, tt_hbm, out_hbm, rowb, colb, outb, semr, semo):
    c = lax.axis_index("c")
    s = lax.axis_index("s")
    w = s * _NC + c
    r0 = w * _RPW

    def fetch(r):
        f = r // _EMB
        e = lax.rem(r, _EMB)
        pltpu.async_copy(tt_hbm.at[f].at[e], rowb, semr)

    f_first = r0 // _EMB
    fetch(r0)
    pltpu.sync_copy(xt_hbm.at[f_first], colb)

    def do_r(j, prev_f):
        r = r0 + j
        f = r // _EMB

        @pl.when(f != prev_f)
        def _():
            pltpu.sync_copy(xt_hbm.at[f], colb)

        pltpu.make_async_copy(tt_hbm.at[0].at[0], rowb, semr).wait()

        @pl.when(j > 0)
        def _():
            pltpu.make_async_copy(outb, out_hbm.at[0], semo).wait()

        def chunk(i, cc):
            idx = colb[pl.ds(i * _L, _L)]
            outb[pl.ds(i * _L, _L)] = plsc.load_gather(rowb, [idx])
            return cc
        lax.fori_loop(0, _BATCH // _L, chunk, 0, unroll=8)

        @pl.when(j + 1 < _RPW)
        def _():
            fetch(r + 1)

        pltpu.async_copy(outb, out_hbm.at[r], semo)
        return f

    lax.fori_loop(0, _RPW, do_r, f_first)
    pltpu.make_async_copy(outb, out_hbm.at[0], semo).wait()

    @pl.when(w < _N_CONT)
    def _():
        pltpu.sync_copy(xt_hbm.at[_N_CAT + w], colb)

        def chunk(i, cc):
            outb[pl.ds(i * _L, _L)] = colb[pl.ds(i * _L, _L)].astype(jnp.float32)
            return cc
        lax.fori_loop(0, _BATCH // _L, chunk, 0, unroll=8)
        pltpu.sync_copy(outb, out_hbm.at[_TAB_ROWS + w])


_emb_call = functools.partial(
    pl.kernel,
    mesh=plsc.VectorSubcoreMesh(core_axis_name="c", subcore_axis_name="s"),
    out_type=jax.ShapeDtypeStruct((_OUT_ROWS, _BATCH), jnp.float32),
    compiler_params=pltpu.CompilerParams(needs_layout_passes=False,
                                         use_tc_tiling_on_sc=True),
    scratch_types=[
        pltpu.VMEM((_VOCAB,), jnp.float32),
        pltpu.VMEM((_BATCH,), jnp.int32),
        pltpu.VMEM((_BATCH,), jnp.float32),
        pltpu.SemaphoreType.DMA,
        pltpu.SemaphoreType.DMA,
    ],
)(_body)


def kernel(x, tables):
    xt = x.T
    tt = tables.transpose(0, 2, 1)
    out_t = _emb_call(xt, tt)
    return out_t.T


# speedup vs baseline: 1.2332x; 1.0199x over previous
import functools

import jax
import jax.numpy as jnp
from jax import lax
from jax.experimental import pallas as pl
from jax.experimental.pallas import tpu as pltpu
from jax.experimental.pallas import tpu_sc as plsc

_INPUT_DIM = 30
_N_CAT = 26
_VOCAB = 100000
_EMB = 16
_BATCH = 4096
_N_CONT = _INPUT_DIM - _N_CAT
_TAB_ROWS = _N_CAT * _EMB
_OUT_ROWS = _TAB_ROWS + _N_CONT
_NC = 2
_NS = 16
_NW = _NC * _NS
_RPW = _TAB_ROWS // _NW
_L = 16


def _body(xt_hbm, tt_hbm, out_hbm, rowb, colb, outb, semr, semo):
    c = lax.axis_index("c")
    s = lax.axis_index("s")
    w = s * _NC + c
    r0 = w * _RPW

    def fetch(r):
        f = r // _EMB
        e = lax.rem(r, _EMB)
        pltpu.async_copy(tt_hbm.at[f].at[e], rowb, semr)

    f_first = r0 // _EMB
    fetch(r0)

    @pl.when(w < _N_CONT)
    def _():
        pltpu.sync_copy(xt_hbm.at[_N_CAT + w], colb)

        def chunk(i, cc):
            outb[pl.ds(i * _L, _L)] = colb[pl.ds(i * _L, _L)].astype(jnp.float32)
            return cc
        lax.fori_loop(0, _BATCH // _L, chunk, 0, unroll=16)
        pltpu.sync_copy(outb, out_hbm.at[_TAB_ROWS + w])

    pltpu.sync_copy(xt_hbm.at[f_first], colb)

    def do_r(j, prev_f):
        r = r0 + j
        f = r // _EMB

        @pl.when(f != prev_f)
        def _():
            pltpu.sync_copy(xt_hbm.at[f], colb)

        pltpu.make_async_copy(tt_hbm.at[0].at[0], rowb, semr).wait()

        @pl.when(j > 0)
        def _():
            pltpu.make_async_copy(outb, out_hbm.at[0], semo).wait()

        def chunk(i, cc):
            idx = colb[pl.ds(i * _L, _L)]
            outb[pl.ds(i * _L, _L)] = plsc.load_gather(rowb, [idx])
            return cc
        lax.fori_loop(0, _BATCH // _L, chunk, 0, unroll=16)

        @pl.when(j + 1 < _RPW)
        def _():
            fetch(r + 1)

        pltpu.async_copy(outb, out_hbm.at[r], semo)
        return f

    lax.fori_loop(0, _RPW, do_r, f_first)
    pltpu.make_async_copy(outb, out_hbm.at[0], semo).wait()


_emb_call = functools.partial(
    pl.kernel,
    mesh=plsc.VectorSubcoreMesh(core_axis_name="c", subcore_axis_name="s"),
    out_type=jax.ShapeDtypeStruct((_OUT_ROWS, _BATCH), jnp.float32),
    compiler_params=pltpu.CompilerParams(needs_layout_passes=False,
                                         use_tc_tiling_on_sc=True),
    scratch_types=[
        pltpu.VMEM((_VOCAB,), jnp.float32),
        pltpu.VMEM((_BATCH,), jnp.int32),
        pltpu.VMEM((_BATCH,), jnp.float32),
        pltpu.SemaphoreType.DMA,
        pltpu.SemaphoreType.DMA,
    ],
)(_body)


def kernel(x, tables):
    xt = x.T
    tt = tables.transpose(0, 2, 1)
    out_t = _emb_call(xt, tt)
    return out_t.T
